# baseline (speedup 1.0000x reference)
import functools

import jax
import jax.numpy as jnp
from jax import lax
from jax.experimental import pallas as pl
from jax.experimental.pallas import tpu as pltpu
from jax.experimental.pallas import tpu_sc as plsc

_ENT = 1000000
_D = 32
_B = 16384
_K = 20
_BK = _B * _K
_NW = 32
_CH = 128
_POSW = _B // _NW
_NEGW = _BK // _NW


def _ids_mask(labels):
    labels = labels.astype(jnp.int32)
    is_ent = labels < _ENT
    return jnp.where(is_ent, labels, labels - _ENT), is_ent


def _rank_by_group(g):
    oh = (g[:, None] == jnp.arange(4, dtype=jnp.int32)).astype(jnp.int32)
    incl = jnp.cumsum(oh, axis=0)
    totals = incl[-1]
    offs = jnp.concatenate([jnp.zeros((1,), jnp.int32), jnp.cumsum(totals)[:-1]])
    return offs[g] + jnp.take_along_axis(incl, g[:, None], axis=1)[:, 0] - 1


def _sorted(x, rank):
    return jnp.zeros_like(x).at[rank].set(x)


def _body(pg, pae, par, pce, pcr, pu, pw,
          ng, nae, nar, nce, ncr, nu, nw,
          t_ie, t_ir, t_oe, t_or, t_mi, t_mo,
          out_hbm,
          idx_g, idx_ae, idx_ar, idx_ce, idx_cr, idx_u, idx_w,
          r_ae, r_ar, r_ce, r_cr, r_u, r_w,
          nls, pls, acc_out, sem):
    info = plsc.get_sparse_core_info()
    wid = lax.axis_index("s") * info.num_cores + lax.axis_index("c")

    def process(src_g, src_ae, src_ar, src_ce, src_cr, src_u, src_w,
                base, out_ref, out_off, sign):
        pltpu.sync_copy(src_g.at[pl.ds(base, _CH)], idx_g)
        pltpu.sync_copy(src_ae.at[pl.ds(base, _CH)], idx_ae)
        pltpu.sync_copy(src_ar.at[pl.ds(base, _CH)], idx_ar)
        pltpu.sync_copy(src_ce.at[pl.ds(base, _CH)], idx_ce)
        pltpu.sync_copy(src_cr.at[pl.ds(base, _CH)], idx_cr)
        pltpu.sync_copy(src_u.at[pl.ds(base, _CH)], idx_u)
        pltpu.sync_copy(src_w.at[pl.ds(base, _CH)], idx_w)
        pltpu.async_copy(t_ie.at[idx_ae], r_ae, sem).wait()
        pltpu.async_copy(t_ir.at[idx_ar], r_ar, sem).wait()
        pltpu.async_copy(t_oe.at[idx_ce], r_ce, sem).wait()
        pltpu.async_copy(t_or.at[idx_cr], r_cr, sem).wait()
        pltpu.async_copy(t_mi.at[idx_u], r_u, sem).wait()
        pltpu.async_copy(t_mo.at[idx_w], r_w, sem).wait()

        def group_body(t, _):
            lane = lax.iota(jnp.int32, 16)
            li = t * 16 + lane
            g = idx_g[pl.ds(t * 16, 16)]
            a_is_ent = g < 2
            c_is_ent = (g == 0) | (g == 2)
            has_u = g == 1
            has_w = g == 2
            def d_body(d, accs):
                sac, sam, scm, smm, saw, scw, sww = accs
                dv = jnp.zeros((16,), jnp.int32) + d
                ae = plsc.load_gather(r_ae, [li, dv])
                ar = plsc.load_gather(r_ar, [li, dv])
                ce_ = plsc.load_gather(r_ce, [li, dv])
                cr_ = plsc.load_gather(r_cr, [li, dv])
                u = plsc.load_gather(r_u, [li, dv])
                w = plsc.load_gather(r_w, [li, dv])
                a = jnp.where(a_is_ent, ae, ar)
                c = jnp.where(c_is_ent, ce_, cr_)
                return (sac + a * c, sam + a * u, scm + c * u, smm + u * u,
                        saw + a * w, scw + c * w, sww + w * w)

            z = jnp.zeros((16,), jnp.float32)
            sac, sam, scm, smm, saw, scw, sww = lax.fori_loop(
                0, _D, d_body, (z, z, z, z, z, z, z))
            corr_u = jnp.where(has_u, sam * scm / jnp.maximum(smm, 1e-24), 0.0)
            corr_w = jnp.where(has_w, saw * scw / jnp.maximum(sww, 1e-24), 0.0)
            x = sign * (sac - corr_u - corr_w)
            e = jnp.exp(-jnp.abs(x))
            tq = e / (2.0 + e)
            t2 = tq * tq
            l1p = 2.0 * tq * (1.0 + t2 * (
                0.3333333333 + t2 * (0.2 + t2 * (0.1428571429 + t2 * 0.1111111111))))
            out_ref[pl.ds(out_off + t * 16, 16)] = jnp.minimum(x, 0.0) - l1p
            return 0

        lax.fori_loop(0, _CH // 16, group_body, 0)

    nbase = wid * _NEGW

    def neg_chunk(ch, _):
        process(ng, nae, nar, nce, ncr, nu, nw,
                nbase + ch * _CH, nls, ch * _CH, -1.0)
        return 0

    lax.fori_loop(0, _NEGW // _CH, neg_chunk, 0)

    pbase = wid * _POSW

    def pos_chunk(ch, _):
        process(pg, pae, par, pce, pcr, pu, pw,
                pbase + ch * _CH, pls, ch * _CH, 1.0)
        return 0

    lax.fori_loop(0, _POSW // _CH, pos_chunk, 0)

    def sum_body(t, _):
        lane = lax.iota(jnp.int32, 16)
        base20 = (t * 16 + lane) * _K

        def k_body(k, acc):
            return acc + plsc.load_gather(nls, [base20 + k])

        acc = lax.fori_loop(0, _K, k_body, jnp.zeros((16,), jnp.float32))
        p = pls[pl.ds(t * 16, 16)]
        acc_out[pl.ds(t * 16, 16)] = -(p + acc)
        return 0

    lax.fori_loop(0, _POSW // 16, sum_body, 0)
    pltpu.sync_copy(acc_out, out_hbm.at[pl.ds(pbase, _POSW)])


def kernel(input_labels, pos_labels, neg_labels,
           in_embed_ent, out_embed_ent, in_embed_rel, out_embed_rel,
           in_embed_map, out_embed_map, reverse_id_map, ent_dic_mask):
    del reverse_id_map, ent_dic_mask
    iid, ei = _ids_mask(input_labels)
    pid, ep = _ids_mask(pos_labels)
    nid, en = _ids_mask(neg_labels.reshape(-1))

    zero = jnp.zeros((), jnp.int32)

    g_pos = 2 * (~ei).astype(jnp.int32) + (~ep).astype(jnp.int32)
    rk = _rank_by_group(g_pos)
    pg = _sorted(g_pos, rk)
    pae = _sorted(jnp.where(ei, iid, zero), rk)
    par = _sorted(jnp.where(ei, zero, iid), rk)
    pce = _sorted(jnp.where(ep, pid, zero), rk)
    pcr = _sorted(jnp.where(ep, zero, pid), rk)
    pu = _sorted(jnp.where(g_pos == 1, pid, zero), rk)
    pw = _sorted(jnp.where(g_pos == 2, iid, zero), rk)

    iid_b = jnp.repeat(iid, _K)
    ei_b = jnp.repeat(ei, _K)
    g_neg = 2 * (~ei_b).astype(jnp.int32) + (~en).astype(jnp.int32)
    rkn = _rank_by_group(g_neg)
    ng = _sorted(g_neg, rkn)
    nae = _sorted(jnp.where(ei_b, iid_b, zero), rkn)
    nar = _sorted(jnp.where(ei_b, zero, iid_b), rkn)
    nce = _sorted(jnp.where(en, nid, zero), rkn)
    ncr = _sorted(jnp.where(en, zero, nid), rkn)
    nu = _sorted(jnp.where(g_neg == 1, nid, zero), rkn)
    nw = _sorted(jnp.where(g_neg == 2, iid_b, zero), rkn)

    mesh = plsc.VectorSubcoreMesh(core_axis_name="c", subcore_axis_name="s")
    run = functools.partial(
        pl.kernel, mesh=mesh,
        out_type=jax.ShapeDtypeStruct((_B,), jnp.float32),
        compiler_params=pltpu.CompilerParams(
            needs_layout_passes=False, use_tc_tiling_on_sc=False),
        scratch_types=(
            [pltpu.VMEM((_CH,), jnp.int32)] * 7
            + [pltpu.VMEM((_CH, _D), jnp.float32)] * 6
            + [pltpu.VMEM((_NEGW,), jnp.float32),
               pltpu.VMEM((_POSW,), jnp.float32),
               pltpu.VMEM((_POSW,), jnp.float32),
               pltpu.SemaphoreType.DMA]),
    )(_body)
    return run(pg, pae, par, pce, pcr, pu, pw,
               ng, nae, nar, nce, ncr, nu, nw,
               in_embed_ent, in_embed_rel, out_embed_ent, out_embed_rel,
               in_embed_map, out_embed_map)


# speedup vs baseline: 1.0037x; 1.0037x over previous
import functools

import jax
import jax.numpy as jnp
from jax import lax
from jax.experimental import pallas as pl
from jax.experimental.pallas import tpu as pltpu
from jax.experimental.pallas import tpu_sc as plsc

_ENT = 1000000
_D = 32
_B = 16384
_K = 20
_BK = _B * _K
_NW = 32
_CH = 128
_POSW = _B // _NW
_NEGW = _BK // _NW
_NT = 6


def _ids_mask(labels):
    labels = labels.astype(jnp.int32)
    is_ent = labels < _ENT
    return jnp.where(is_ent, labels, labels - _ENT), is_ent


def _rank_by_group(g):
    oh = (g[:, None] == jnp.arange(4, dtype=jnp.int32)).astype(jnp.int32)
    incl = jnp.cumsum(oh, axis=0)
    totals = incl[-1]
    offs = jnp.concatenate([jnp.zeros((1,), jnp.int32), jnp.cumsum(totals)[:-1]])
    return offs[g] + jnp.take_along_axis(incl, g[:, None], axis=1)[:, 0] - 1


def _sorted(x, rank):
    return jnp.zeros_like(x).at[rank].set(x)


def _pack_chunks(arrs, rank):
    s = jnp.stack([_sorted(a, rank) for a in arrs])
    n = s.shape[1]
    return s.reshape(_NT, n // _CH, _CH).transpose(1, 0, 2)


def _body(ppk, pg, npk, ng,
          t_ie, t_ir, t_oe, t_or, t_mi, t_mo,
          out_hbm,
          pk0, g0, r0a, r0b, r0c, r0d, r0e, r0f, sem0,
          pk1, g1, r1a, r1b, r1c, r1d, r1e, r1f, sem1,
          nls, pls, acc_out):
    info = plsc.get_sparse_core_info()
    wid = lax.axis_index("s") * info.num_cores + lax.axis_index("c")
    tables = (t_ie, t_ir, t_oe, t_or, t_mi, t_mo)
    slot0 = (pk0, g0, (r0a, r0b, r0c, r0d, r0e, r0f), sem0)
    slot1 = (pk1, g1, (r1a, r1b, r1c, r1d, r1e, r1f), sem1)

    def stage(src_pk, src_g, cid, slot):
        pk, gbuf, rows, sem = slot
        pltpu.sync_copy(src_pk.at[cid], pk)
        pltpu.sync_copy(src_g.at[pl.ds(cid * _CH, _CH)], gbuf)
        for j in range(_NT):
            pltpu.async_copy(tables[j].at[pk.at[j]], rows[j], sem)

    def drain_compute(slot, out_ref, out_off, sign):
        pk, gbuf, rows, sem = slot
        for j in range(_NT):
            pltpu.make_async_copy(tables[j].at[pk.at[j]], rows[j], sem).wait()
        r_ae, r_ar, r_ce, r_cr, r_u, r_w = rows

        def group_body(t, _):
            lane = lax.iota(jnp.int32, 16)
            li = t * 16 + lane
            g = gbuf[pl.ds(t * 16, 16)]
            a_is_ent = g < 2
            c_is_ent = (g == 0) | (g == 2)
            has_u = g == 1
            has_w = g == 2

            def d_body(d, accs):
                sac, sam, scm, smm, saw, scw, sww = accs
                dv = jnp.zeros((16,), jnp.int32) + d
                ae = plsc.load_gather(r_ae, [li, dv])
                ar = plsc.load_gather(r_ar, [li, dv])
                ce_ = plsc.load_gather(r_ce, [li, dv])
                cr_ = plsc.load_gather(r_cr, [li, dv])
                u = plsc.load_gather(r_u, [li, dv])
                w = plsc.load_gather(r_w, [li, dv])
                a = jnp.where(a_is_ent, ae, ar)
                c = jnp.where(c_is_ent, ce_, cr_)
                return (sac + a * c, sam + a * u, scm + c * u, smm + u * u,
                        saw + a * w, scw + c * w, sww + w * w)

            z = jnp.zeros((16,), jnp.float32)
            sac, sam, scm, smm, saw, scw, sww = lax.fori_loop(
                0, _D, d_body, (z, z, z, z, z, z, z))
            corr_u = jnp.where(has_u, sam * scm / jnp.maximum(smm, 1e-24), 0.0)
            corr_w = jnp.where(has_w, saw * scw / jnp.maximum(sww, 1e-24), 0.0)
            x = sign * (sac - corr_u - corr_w)
            e = jnp.exp(-jnp.abs(x))
            tq = e / (2.0 + e)
            t2 = tq * tq
            l1p = 2.0 * tq * (1.0 + t2 * (
                0.3333333333 + t2 * (0.2 + t2 * (0.1428571429 + t2 * 0.1111111111))))
            out_ref[pl.ds(out_off + t * 16, 16)] = jnp.minimum(x, 0.0) - l1p
            return 0

        lax.fori_loop(0, _CH // 16, group_body, 0)

    def run_phase(src_pk, src_g, cid0, nch, out_ref, sign):
        stage(src_pk, src_g, cid0, slot0)

        def body(h, _):
            c0 = 2 * h
            stage(src_pk, src_g, cid0 + c0 + 1, slot1)
            drain_compute(slot0, out_ref, c0 * _CH, sign)

            @pl.when(h + 1 < nch // 2)
            def _():
                stage(src_pk, src_g, cid0 + c0 + 2, slot0)

            drain_compute(slot1, out_ref, (c0 + 1) * _CH, sign)
            return 0

        lax.fori_loop(0, nch // 2, body, 0)

    nch_n = _NEGW // _CH
    run_phase(npk, ng, wid * nch_n, nch_n, nls, -1.0)
    nch_p = _POSW // _CH
    run_phase(ppk, pg, wid * nch_p, nch_p, pls, 1.0)

    def sum_body(t, _):
        lane = lax.iota(jnp.int32, 16)
        base20 = (t * 16 + lane) * _K

        def k_body(k, acc):
            return acc + plsc.load_gather(nls, [base20 + k])

        acc = lax.fori_loop(0, _K, k_body, jnp.zeros((16,), jnp.float32))
        p = pls[pl.ds(t * 16, 16)]
        acc_out[pl.ds(t * 16, 16)] = -(p + acc)
        return 0

    lax.fori_loop(0, _POSW // 16, sum_body, 0)
    pltpu.sync_copy(acc_out, out_hbm.at[pl.ds(wid * _POSW, _POSW)])


def kernel(input_labels, pos_labels, neg_labels,
           in_embed_ent, out_embed_ent, in_embed_rel, out_embed_rel,
           in_embed_map, out_embed_map, reverse_id_map, ent_dic_mask):
    del reverse_id_map, ent_dic_mask
    iid, ei = _ids_mask(input_labels)
    pid, ep = _ids_mask(pos_labels)
    nid, en = _ids_mask(neg_labels.reshape(-1))

    zero = jnp.zeros((), jnp.int32)

    g_pos = 2 * (~ei).astype(jnp.int32) + (~ep).astype(jnp.int32)
    rk = _rank_by_group(g_pos)
    pg = _sorted(g_pos, rk)
    ppk = _pack_chunks(
        [jnp.where(ei, iid, zero), jnp.where(ei, zero, iid),
         jnp.where(ep, pid, zero), jnp.where(ep, zero, pid),
         jnp.where(g_pos == 1, pid, zero), jnp.where(g_pos == 2, iid, zero)],
        rk)

    iid_b = jnp.repeat(iid, _K)
    ei_b = jnp.repeat(ei, _K)
    g_neg = 2 * (~ei_b).astype(jnp.int32) + (~en).astype(jnp.int32)
    rkn = _rank_by_group(g_neg)
    ng = _sorted(g_neg, rkn)
    npk = _pack_chunks(
        [jnp.where(ei_b, iid_b, zero), jnp.where(ei_b, zero, iid_b),
         jnp.where(en, nid, zero), jnp.where(en, zero, nid),
         jnp.where(g_neg == 1, nid, zero), jnp.where(g_neg == 2, iid_b, zero)],
        rkn)

    mesh = plsc.VectorSubcoreMesh(core_axis_name="c", subcore_axis_name="s")
    slot = [pltpu.VMEM((_NT, _CH), jnp.int32), pltpu.VMEM((_CH,), jnp.int32)] \
        + [pltpu.VMEM((_CH, _D), jnp.float32)] * _NT + [pltpu.SemaphoreType.DMA]
    run = functools.partial(
        pl.kernel, mesh=mesh,
        out_type=jax.ShapeDtypeStruct((_B,), jnp.float32),
        compiler_params=pltpu.CompilerParams(
            needs_layout_passes=False, use_tc_tiling_on_sc=False),
        scratch_types=(
            slot + slot
            + [pltpu.VMEM((_NEGW,), jnp.float32),
               pltpu.VMEM((_POSW,), jnp.float32),
               pltpu.VMEM((_POSW,), jnp.float32)]),
    )(_body)
    return run(ppk, pg, npk, ng,
               in_embed_ent, in_embed_rel, out_embed_ent, out_embed_rel,
               in_embed_map, out_embed_map)


# speedup vs baseline: 1.4463x; 1.4410x over previous
import functools

import jax
import jax.numpy as jnp
from jax import lax
from jax.experimental import pallas as pl
from jax.experimental.pallas import tpu as pltpu
from jax.experimental.pallas import tpu_sc as plsc

_ENT = 1000000
_D = 32
_B = 16384
_K = 20
_BK = _B * _K
_NW = 32
_CH = 128
_POSW = _B // _NW
_NEGW = _BK // _NW
_NT = 6


def _ids_mask(labels):
    labels = labels.astype(jnp.int32)
    is_ent = labels < _ENT
    return jnp.where(is_ent, labels, labels - _ENT), is_ent


def _rank_by_group(g):
    oh = (g[:, None] == jnp.arange(4, dtype=jnp.int32)).astype(jnp.int32)
    incl = jnp.cumsum(oh, axis=0)
    totals = incl[-1]
    offs = jnp.concatenate([jnp.zeros((1,), jnp.int32), jnp.cumsum(totals)[:-1]])
    return offs[g] + jnp.take_along_axis(incl, g[:, None], axis=1)[:, 0] - 1


def _perm_of_rank(rank):
    n = rank.shape[0]
    return jnp.zeros((n,), jnp.int32).at[rank].set(
        jnp.arange(n, dtype=jnp.int32), unique_indices=True)


def _pack_chunks(arrs, perm):
    s = jnp.stack(arrs, axis=1)[perm]
    n = s.shape[0]
    return s.reshape(n // _CH, _CH, _NT).transpose(0, 2, 1)


def _body(ppk, pg, npk, ng,
          t_ie, t_ir, t_oe, t_or, t_mi, t_mo,
          out_hbm,
          pk0, g0, r0a, r0b, r0c, r0d, r0e, r0f, sem0,
          pk1, g1, r1a, r1b, r1c, r1d, r1e, r1f, sem1,
          nls, pls, acc_out):
    info = plsc.get_sparse_core_info()
    wid = lax.axis_index("s") * info.num_cores + lax.axis_index("c")
    tables = (t_ie, t_ir, t_oe, t_or, t_mi, t_mo)
    slot0 = (pk0, g0, (r0a, r0b, r0c, r0d, r0e, r0f), sem0)
    slot1 = (pk1, g1, (r1a, r1b, r1c, r1d, r1e, r1f), sem1)

    def stage(src_pk, src_g, cid, slot):
        pk, gbuf, rows, sem = slot
        pltpu.sync_copy(src_pk.at[cid], pk)
        pltpu.sync_copy(src_g.at[pl.ds(cid * _CH, _CH)], gbuf)
        for j in range(_NT):
            pltpu.async_copy(tables[j].at[pk.at[j]], rows[j], sem)

    def drain_compute(slot, out_ref, out_off, sign):
        pk, gbuf, rows, sem = slot
        for j in range(_NT):
            pltpu.make_async_copy(tables[j].at[pk.at[j]], rows[j], sem).wait()
        r_ae, r_ar, r_ce, r_cr, r_u, r_w = rows

        def group_body(t, _):
            lane = lax.iota(jnp.int32, 16)
            li = t * 16 + lane
            g = gbuf[pl.ds(t * 16, 16)]
            a_is_ent = g < 2
            c_is_ent = (g == 0) | (g == 2)
            has_u = g == 1
            has_w = g == 2

            def d_body(d, accs):
                sac, sam, scm, smm, saw, scw, sww = accs
                dv = jnp.zeros((16,), jnp.int32) + d
                ae = plsc.load_gather(r_ae, [li, dv])
                ar = plsc.load_gather(r_ar, [li, dv])
                ce_ = plsc.load_gather(r_ce, [li, dv])
                cr_ = plsc.load_gather(r_cr, [li, dv])
                u = plsc.load_gather(r_u, [li, dv])
                w = plsc.load_gather(r_w, [li, dv])
                a = jnp.where(a_is_ent, ae, ar)
                c = jnp.where(c_is_ent, ce_, cr_)
                return (sac + a * c, sam + a * u, scm + c * u, smm + u * u,
                        saw + a * w, scw + c * w, sww + w * w)

            z = jnp.zeros((16,), jnp.float32)
            sac, sam, scm, smm, saw, scw, sww = lax.fori_loop(
                0, _D, d_body, (z, z, z, z, z, z, z))
            corr_u = jnp.where(has_u, sam * scm / jnp.maximum(smm, 1e-24), 0.0)
            corr_w = jnp.where(has_w, saw * scw / jnp.maximum(sww, 1e-24), 0.0)
            x = sign * (sac - corr_u - corr_w)
            e = jnp.exp(-jnp.abs(x))
            tq = e / (2.0 + e)
            t2 = tq * tq
            l1p = 2.0 * tq * (1.0 + t2 * (
                0.3333333333 + t2 * (0.2 + t2 * (0.1428571429 + t2 * 0.1111111111))))
            out_ref[pl.ds(out_off + t * 16, 16)] = jnp.minimum(x, 0.0) - l1p
            return 0

        lax.fori_loop(0, _CH // 16, group_body, 0)

    def run_phase(src_pk, src_g, cid0, nch, out_ref, sign):
        stage(src_pk, src_g, cid0, slot0)

        def body(h, _):
            c0 = 2 * h
            stage(src_pk, src_g, cid0 + c0 + 1, slot1)
            drain_compute(slot0, out_ref, c0 * _CH, sign)

            @pl.when(h + 1 < nch // 2)
            def _():
                stage(src_pk, src_g, cid0 + c0 + 2, slot0)

            drain_compute(slot1, out_ref, (c0 + 1) * _CH, sign)
            return 0

        lax.fori_loop(0, nch // 2, body, 0)

    nch_n = _NEGW // _CH
    run_phase(npk, ng, wid * nch_n, nch_n, nls, -1.0)
    nch_p = _POSW // _CH
    run_phase(ppk, pg, wid * nch_p, nch_p, pls, 1.0)

    def sum_body(t, _):
        lane = lax.iota(jnp.int32, 16)
        base20 = (t * 16 + lane) * _K

        def k_body(k, acc):
            return acc + plsc.load_gather(nls, [base20 + k])

        acc = lax.fori_loop(0, _K, k_body, jnp.zeros((16,), jnp.float32))
        p = pls[pl.ds(t * 16, 16)]
        acc_out[pl.ds(t * 16, 16)] = -(p + acc)
        return 0

    lax.fori_loop(0, _POSW // 16, sum_body, 0)
    pltpu.sync_copy(acc_out, out_hbm.at[pl.ds(wid * _POSW, _POSW)])


def kernel(input_labels, pos_labels, neg_labels,
           in_embed_ent, out_embed_ent, in_embed_rel, out_embed_rel,
           in_embed_map, out_embed_map, reverse_id_map, ent_dic_mask):
    del reverse_id_map, ent_dic_mask
    iid, ei = _ids_mask(input_labels)
    pid, ep = _ids_mask(pos_labels)
    nid, en = _ids_mask(neg_labels.reshape(-1))

    zero = jnp.zeros((), jnp.int32)

    g_pos = 2 * (~ei).astype(jnp.int32) + (~ep).astype(jnp.int32)
    pperm = _perm_of_rank(_rank_by_group(g_pos))
    pg = g_pos[pperm]
    ppk = _pack_chunks(
        [jnp.where(ei, iid, zero), jnp.where(ei, zero, iid),
         jnp.where(ep, pid, zero), jnp.where(ep, zero, pid),
         jnp.where(g_pos == 1, pid, zero), jnp.where(g_pos == 2, iid, zero)],
        pperm)

    iid_b = jnp.repeat(iid, _K)
    ei_b = jnp.repeat(ei, _K)
    g_neg = 2 * (~ei_b).astype(jnp.int32) + (~en).astype(jnp.int32)
    nperm = _perm_of_rank(_rank_by_group(g_neg))
    ng = g_neg[nperm]
    npk = _pack_chunks(
        [jnp.where(ei_b, iid_b, zero), jnp.where(ei_b, zero, iid_b),
         jnp.where(en, nid, zero), jnp.where(en, zero, nid),
         jnp.where(g_neg == 1, nid, zero), jnp.where(g_neg == 2, iid_b, zero)],
        nperm)

    mesh = plsc.VectorSubcoreMesh(core_axis_name="c", subcore_axis_name="s")
    slot = [pltpu.VMEM((_NT, _CH), jnp.int32), pltpu.VMEM((_CH,), jnp.int32)] \
        + [pltpu.VMEM((_CH, _D), jnp.float32)] * _NT + [pltpu.SemaphoreType.DMA]
    run = functools.partial(
        pl.kernel, mesh=mesh,
        out_type=jax.ShapeDtypeStruct((_B,), jnp.float32),
        compiler_params=pltpu.CompilerParams(
            needs_layout_passes=False, use_tc_tiling_on_sc=False),
        scratch_types=(
            slot + slot
            + [pltpu.VMEM((_NEGW,), jnp.float32),
               pltpu.VMEM((_POSW,), jnp.float32),
               pltpu.VMEM((_POSW,), jnp.float32)]),
    )(_body)
    return run(ppk, pg, npk, ng,
               in_embed_ent, in_embed_rel, out_embed_ent, out_embed_rel,
               in_embed_map, out_embed_map)
